# baseline (speedup 1.0000x reference)
import jax
import jax.numpy as jnp
from jax.experimental import pallas as pl
from jax.experimental.pallas import tpu as pltpu

_CHUNKS = (128, 128, 256) + (512,) * 15
_N_BIG = 4


def _router_body(x_hbm, w_ref, noise_ref, out_ref, s0, s1, m0, b0, b1, b2, b3, *sems):
    bigs = (b0, b1, b2, b3)
    offs = []
    o = 0
    for c in _CHUNKS:
        offs.append(o)
        o += c

    def buf(i):
        if i == 0:
            return s0
        if i == 1:
            return s1
        if i == 2:
            return m0
        return bigs[(i - 3) % _N_BIG]

    def copy(i):
        return pltpu.make_async_copy(
            x_hbm.at[pl.ds(offs[i], _CHUNKS[i]), :], buf(i), sems[i]
        )

    n = len(_CHUNKS)
    for i in range(3 + _N_BIG):
        copy(i).start()
    for i in range(n):
        copy(i).wait()
        rows = _CHUNKS[i]
        acc = jax.lax.dot_general(
            buf(i)[...],
            w_ref[...],
            dimension_numbers=(((1,), (1,)), ((), ())),
            preferred_element_type=jnp.float32,
        )
        out_ref[pl.ds(offs[i], rows), :] = acc + noise_ref[pl.ds(offs[i], rows), :]
        if i + 3 + _N_BIG < n:
            copy(i + 3 + _N_BIG).start()


def kernel(x, W, noise):
    tokens, d_model = x.shape
    n_experts = W.shape[0]
    return pl.pallas_call(
        _router_body,
        in_specs=[
            pl.BlockSpec(memory_space=pltpu.MemorySpace.HBM),
            pl.BlockSpec(memory_space=pltpu.MemorySpace.VMEM),
            pl.BlockSpec(memory_space=pltpu.MemorySpace.VMEM),
        ],
        out_specs=pl.BlockSpec(memory_space=pltpu.MemorySpace.VMEM),
        out_shape=jax.ShapeDtypeStruct((tokens, n_experts), jnp.float32),
        scratch_shapes=[
            pltpu.VMEM((128, d_model), jnp.float32),
            pltpu.VMEM((128, d_model), jnp.float32),
            pltpu.VMEM((256, d_model), jnp.float32),
        ]
        + [pltpu.VMEM((512, d_model), jnp.float32) for _ in range(_N_BIG)]
        + [pltpu.SemaphoreType.DMA for _ in range(len(_CHUNKS))],
    )(x, W, noise)


# speedup vs baseline: 1.0932x; 1.0932x over previous
import jax
import jax.numpy as jnp
from jax.experimental import pallas as pl
from jax.experimental.pallas import tpu as pltpu


def _router_block(x_ref, w_ref, noise_ref, out_ref):
    i = pl.program_id(0)
    blk = x_ref.shape[0]
    acc = jax.lax.dot_general(
        x_ref[...],
        w_ref[...],
        dimension_numbers=(((1,), (1,)), ((), ())),
        preferred_element_type=jnp.float32,
    )
    out_ref[...] = acc + noise_ref[pl.ds(i * blk, blk), :]


def kernel(x, W, noise):
    tokens, d_model = x.shape
    n_experts = W.shape[0]
    blk = 512
    return pl.pallas_call(
        _router_block,
        grid=(tokens // blk,),
        in_specs=[
            pl.BlockSpec((blk, d_model), lambda i: (i, 0)),
            pl.BlockSpec((n_experts, d_model), lambda i: (0, 0)),
            pl.BlockSpec((tokens, n_experts), lambda i: (0, 0)),
        ],
        out_specs=pl.BlockSpec((blk, n_experts), lambda i: (i, 0)),
        out_shape=jax.ShapeDtypeStruct((tokens, n_experts), jnp.float32),
        compiler_params=pltpu.CompilerParams(
            dimension_semantics=("arbitrary",),
        ),
    )(x, W, noise)
